# Initial kernel scaffold; baseline (speedup 1.0000x reference)
import jax
import jax.numpy as jnp
from jax.experimental import pallas as pl


def kernel(x, cos, sin, ln1_w, ln1_b, ln2_w, ln2_b, Wq, bq, Wk, bk, Wv, bv, Wo, bo, Wg, W1, W2, W3):
    raise NotImplementedError("write your pallas kernel here")


# speedup vs baseline: 1.1470x; 1.1470x over previous
import functools

import numpy as np
import jax
import jax.numpy as jnp
from jax.experimental import pallas as pl
from jax.experimental.pallas import tpu as pltpu

B, L, D = 1, 2048, 1024
H, KVH, HD = 16, 4, 64
E, K, F = 8, 2, 2048
EPS = 1e-05
G = H // KVH

BT = 128
NPAD = K * L + E * BT
NB = NPAD // BT

_P = np.zeros((HD, HD), np.float32)
for _i in range(HD // 2):
    _P[HD // 2 + _i, 2 * _i] = -1.0
    _P[_i, 2 * _i + 1] = 1.0
_BDQ = np.kron(np.eye(H, dtype=np.float32), _P)
_BDK = np.kron(np.eye(KVH, dtype=np.float32), _P)


def _rmsnorm(xb, w, b):
    rms = jax.lax.rsqrt(jnp.mean(xb * xb, axis=-1, keepdims=True) + EPS)
    return w * (xb * rms) + b


def _preattn_body(x_ref, ln1w_ref, ln1b_ref, wq_ref, bq_ref, wk_ref, bk_ref,
                  wv_ref, bv_ref, cq_ref, sq_ref, ck_ref, sk_ref,
                  bdq_ref, bdk_ref, q_ref, k_ref, v_ref):
    h = _rmsnorm(x_ref[...], ln1w_ref[...], ln1b_ref[...])
    dn = (((1,), (1,)), ((), ()))
    q = jax.lax.dot_general(h, wq_ref[...], dn,
                            preferred_element_type=jnp.float32) + bq_ref[...]
    k = jax.lax.dot_general(h, wk_ref[...], dn,
                            preferred_element_type=jnp.float32) + bk_ref[...]
    v = jax.lax.dot_general(h, wv_ref[...], dn,
                            preferred_element_type=jnp.float32) + bv_ref[...]
    dnn = (((1,), (0,)), ((), ()))
    qr = q * cq_ref[...] + jax.lax.dot_general(
        q, bdq_ref[...], dnn, preferred_element_type=jnp.float32) * sq_ref[...]
    kr = k * ck_ref[...] + jax.lax.dot_general(
        k, bdk_ref[...], dnn, preferred_element_type=jnp.float32) * sk_ref[...]
    q_ref[...] = qr
    k_ref[...] = kr
    v_ref[...] = v


def _attn_body(q_ref, k_ref, v_ref, o_ref, *, bq):
    i = pl.program_id(1)
    q = q_ref[0]
    k = k_ref[0]
    v = v_ref[0]
    s = jax.lax.dot_general(q, k, (((1,), (1,)), ((), ())),
                            preferred_element_type=jnp.float32) * (1.0 / 8.0)
    rows = jax.lax.broadcasted_iota(jnp.int32, (bq, L), 0) + i * bq
    cols = jax.lax.broadcasted_iota(jnp.int32, (bq, L), 1)
    s = jnp.where(rows >= cols, s, jnp.float32(-1e30))
    m = jnp.max(s, axis=-1, keepdims=True)
    p = jnp.exp(s - m)
    l = jnp.sum(p, axis=-1, keepdims=True)
    o = jax.lax.dot_general(p, v, (((1,), (0,)), ((), ())),
                            preferred_element_type=jnp.float32) / l
    o_ref[0] = o


def _postattn_body(o_ref, res_ref, wo_ref, bo_ref, ln2w_ref, ln2b_ref,
                   wg_ref, x2_ref, h2_ref, rl_ref, ti_ref, tw_ref):
    dn = (((1,), (1,)), ((), ()))
    x2 = jax.lax.dot_general(o_ref[...], wo_ref[...], dn,
                             preferred_element_type=jnp.float32) \
        + bo_ref[...] + res_ref[...]
    h2 = _rmsnorm(x2, ln2w_ref[...], ln2b_ref[...])
    rl = jax.lax.dot_general(h2, wg_ref[...], dn,
                             preferred_element_type=jnp.float32)
    bt = rl.shape[0]
    col = jax.lax.broadcasted_iota(jnp.int32, (bt, 128), 1)
    rlm = jnp.where(col < E, rl, jnp.float32(-1e30))
    m1 = jnp.max(rlm, axis=-1, keepdims=True)
    i1 = jnp.min(jnp.where(rlm == m1, col, 127), axis=-1, keepdims=True)
    rl2 = jnp.where(col == i1, jnp.float32(-1e30), rlm)
    m2 = jnp.max(rl2, axis=-1, keepdims=True)
    i2 = jnp.min(jnp.where(rl2 == m2, col, 127), axis=-1, keepdims=True)
    w1 = 1.0 / (1.0 + jnp.exp(m2 - m1))
    w2 = 1.0 - w1
    x2_ref[...] = x2
    h2_ref[...] = h2
    rl_ref[...] = rlm
    ti_ref[...] = jnp.where(col == 0, i1, jnp.where(col == 1, i2, 0))
    tw_ref[...] = jnp.where(col == 0, w1, jnp.where(col == 1, w2, 0.0))


def _moe_body(be_ref, st_ref, h2_ref, w1_ref, w2_ref, w3_ref, ws_ref,
              eo_ref, xb_ref):
    b = pl.program_id(0)
    e = be_ref[b]

    @pl.when(e >= 0)
    def _():
        def cp(r, c):
            t = st_ref[b * BT + r]
            xb_ref[r, :] = h2_ref[t, :]
            return c
        jax.lax.fori_loop(0, BT, cp, 0, unroll=8)
        xb = xb_ref[...]
        dn = (((1,), (1,)), ((), ()))
        h1 = jax.lax.dot_general(xb, w1_ref[0], dn,
                                 preferred_element_type=jnp.float32)
        h3 = jax.lax.dot_general(xb, w3_ref[0], dn,
                                 preferred_element_type=jnp.float32)
        hh = h1 * jax.lax.logistic(h1) * h3
        eo = jax.lax.dot_general(hh, w2_ref[0], dn,
                                 preferred_element_type=jnp.float32)
        eo_ref[...] = eo * ws_ref[:, 0:1]


def _combine_body(p_ref, eo_ref, x2_ref, out_ref, *, btc):
    c = pl.program_id(0)

    def cp(r, acc):
        t = c * btc + r
        pa = p_ref[t]
        pb = p_ref[L + t]
        out_ref[r, :] = x2_ref[r, :] + eo_ref[pa, :] + eo_ref[pb, :]
        return acc
    jax.lax.fori_loop(0, btc, cp, 0, unroll=8)


def kernel(x, cos, sin, ln1_w, ln1_b, ln2_w, ln2_b, Wq, bq, Wk, bk, Wv, bv,
           Wo, bo, Wg, W1, W2, W3):
    xf = x.reshape(L, D)
    cq = jnp.tile(cos, (1, H))
    sq = jnp.tile(sin, (1, H))
    ck = jnp.tile(cos, (1, KVH))
    sk = jnp.tile(sin, (1, KVH))
    bdq = jnp.asarray(_BDQ)
    bdk = jnp.asarray(_BDK)

    bt = 256
    nblk = L // bt
    full = lambda shape: pl.BlockSpec(shape, lambda i: tuple(0 for _ in shape))
    row_blk = lambda w: pl.BlockSpec((bt, w), lambda i: (i, 0))
    q, k, v = pl.pallas_call(
        _preattn_body,
        grid=(nblk,),
        in_specs=[row_blk(D), full((1, D)), full((1, D)),
                  full((H * HD, D)), full((1, H * HD)),
                  full((KVH * HD, D)), full((1, KVH * HD)),
                  full((KVH * HD, D)), full((1, KVH * HD)),
                  row_blk(H * HD), row_blk(H * HD),
                  row_blk(KVH * HD), row_blk(KVH * HD),
                  full((H * HD, H * HD)), full((KVH * HD, KVH * HD))],
        out_specs=[row_blk(H * HD), row_blk(KVH * HD), row_blk(KVH * HD)],
        out_shape=[jax.ShapeDtypeStruct((L, H * HD), jnp.float32),
                   jax.ShapeDtypeStruct((L, KVH * HD), jnp.float32),
                   jax.ShapeDtypeStruct((L, KVH * HD), jnp.float32)],
    )(xf, ln1_w[None], ln1_b[None], Wq, bq[None], Wk, bk[None], Wv, bv[None],
      cq, sq, ck, sk, bdq, bdk)

    q3 = q.reshape(L, H, HD).transpose(1, 0, 2)
    k3 = k.reshape(L, KVH, HD).transpose(1, 0, 2)
    v3 = v.reshape(L, KVH, HD).transpose(1, 0, 2)

    bq_a = 256
    o3 = pl.pallas_call(
        functools.partial(_attn_body, bq=bq_a),
        grid=(H, L // bq_a),
        in_specs=[pl.BlockSpec((1, bq_a, HD), lambda h, i: (h, i, 0)),
                  pl.BlockSpec((1, L, HD), lambda h, i: (h // G, 0, 0)),
                  pl.BlockSpec((1, L, HD), lambda h, i: (h // G, 0, 0))],
        out_specs=pl.BlockSpec((1, bq_a, HD), lambda h, i: (h, i, 0)),
        out_shape=jax.ShapeDtypeStruct((H, L, HD), jnp.float32),
    )(q3, k3, v3)
    o = o3.transpose(1, 0, 2).reshape(L, H * HD)

    wg_pad = jnp.zeros((128, D), jnp.float32).at[:E].set(Wg)
    x2, h2, rl, ti, tw = pl.pallas_call(
        _postattn_body,
        grid=(nblk,),
        in_specs=[row_blk(H * HD), row_blk(D), full((D, H * HD)),
                  full((1, D)), full((1, D)), full((1, D)), full((128, D))],
        out_specs=[row_blk(D), row_blk(D), row_blk(128), row_blk(128),
                   row_blk(128)],
        out_shape=[jax.ShapeDtypeStruct((L, D), jnp.float32),
                   jax.ShapeDtypeStruct((L, D), jnp.float32),
                   jax.ShapeDtypeStruct((L, 128), jnp.float32),
                   jax.ShapeDtypeStruct((L, 128), jnp.int32),
                   jax.ShapeDtypeStruct((L, 128), jnp.float32)],
    )(o, xf, Wo, bo[None], ln2_w[None], ln2_b[None], wg_pad)
    router_logits = rl[:, :E]

    e_flat = jnp.concatenate([ti[:, 0], ti[:, 1]])
    w_flat = jnp.concatenate([tw[:, 0], tw[:, 1]])
    order = jnp.argsort(e_flat, stable=True)
    e_sorted = e_flat[order]
    counts = jnp.sum(e_flat[:, None] == jnp.arange(E)[None, :], axis=0)
    padded = ((counts + BT - 1) // BT) * BT
    pad_start = jnp.concatenate([jnp.zeros((1,), jnp.int32),
                                 jnp.cumsum(padded).astype(jnp.int32)])
    start = jnp.concatenate([jnp.zeros((1,), jnp.int32),
                             jnp.cumsum(counts).astype(jnp.int32)])
    r = jnp.arange(K * L, dtype=jnp.int32) - start[e_sorted]
    p = pad_start[e_sorted] + r
    src_token = jnp.zeros((NPAD,), jnp.int32).at[p].set(
        (order % L).astype(jnp.int32))
    pos_flat = jnp.zeros((K * L,), jnp.int32).at[order].set(p)
    ws = jnp.zeros((NPAD,), jnp.float32).at[p].set(w_flat[order])
    ws128 = jnp.broadcast_to(ws[:, None], (NPAD, 128))
    total_pad = pad_start[E]
    bstart = jnp.arange(NB, dtype=jnp.int32) * BT
    block_expert = jnp.where(
        bstart < total_pad,
        jnp.searchsorted(pad_start, bstart, side='right').astype(jnp.int32) - 1,
        -1)

    eo_buf = pl.pallas_call(
        _moe_body,
        grid_spec=pltpu.PrefetchScalarGridSpec(
            num_scalar_prefetch=2,
            grid=(NB,),
            in_specs=[
                pl.BlockSpec((L, D), lambda b, be, st: (0, 0)),
                pl.BlockSpec((1, F, D),
                             lambda b, be, st: (jnp.maximum(be[b], 0), 0, 0)),
                pl.BlockSpec((1, D, F),
                             lambda b, be, st: (jnp.maximum(be[b], 0), 0, 0)),
                pl.BlockSpec((1, F, D),
                             lambda b, be, st: (jnp.maximum(be[b], 0), 0, 0)),
                pl.BlockSpec((BT, 128), lambda b, be, st: (b, 0)),
            ],
            out_specs=pl.BlockSpec((BT, D), lambda b, be, st: (b, 0)),
            scratch_shapes=[pltpu.VMEM((BT, D), jnp.float32)],
        ),
        out_shape=jax.ShapeDtypeStruct((NPAD, D), jnp.float32),
        compiler_params=pltpu.CompilerParams(
            vmem_limit_bytes=100 * 1024 * 1024),
    )(block_expert, src_token, h2, W1, W2, W3, ws128)

    btc = 256
    xout = pl.pallas_call(
        functools.partial(_combine_body, btc=btc),
        grid_spec=pltpu.PrefetchScalarGridSpec(
            num_scalar_prefetch=1,
            grid=(L // btc,),
            in_specs=[pl.BlockSpec((NPAD, D), lambda c, pf: (0, 0)),
                      pl.BlockSpec((btc, D), lambda c, pf: (c, 0))],
            out_specs=pl.BlockSpec((btc, D), lambda c, pf: (c, 0)),
        ),
        out_shape=jax.ShapeDtypeStruct((L, D), jnp.float32),
        compiler_params=pltpu.CompilerParams(
            vmem_limit_bytes=100 * 1024 * 1024),
    )(pos_flat, eo_buf, x2)

    return (xout.reshape(B, L, D), router_logits)
